# Initial kernel scaffold; baseline (speedup 1.0000x reference)
import jax
import jax.numpy as jnp
from jax.experimental import pallas as pl


def kernel(x, edge_weight, edge_index):
    raise NotImplementedError("write your pallas kernel here")


# speedup vs baseline: 4.2354x; 4.2354x over previous
import jax
import jax.numpy as jnp
from jax import lax
from jax.experimental import pallas as pl
from jax.experimental.pallas import tpu as pltpu
from jax.experimental.pallas import tpu_sc as plsc

N_NODES = 10000
N_EDGES = 320000
D = 128
NC, NS, L = 2, 16, 16
NW = NC * NS
EPW = N_EDGES // NW
CH = 80
NCHUNK = EPW // CH
NPAD = 10240
RPT = NPAD // NS
ZR = 80


def _sc_body(x_hbm, src_hbm, dst_hbm, w_hbm,
             pooled_hbm, deg_hbm,
             idx_s, idx_d, wv, rows, deg, acc, gsem):
    cid = lax.axis_index("c")
    sid = lax.axis_index("s")
    wid = cid * NS + sid

    zeros = jnp.zeros((L,), jnp.float32)
    lane = lax.iota(jnp.int32, L)

    @pl.loop(0, ZR)
    def _zero_rows(r):
        for j in range(D // L):
            rows[r, pl.ds(j * L, L)] = zeros

    @pl.loop(0, NPAD // L)
    def _zero_deg(i):
        deg[pl.ds(i * L, L)] = zeros

    for k in range(RPT // ZR):
        pltpu.sync_copy(rows, acc.at[pl.ds(sid * RPT + k * ZR, ZR)])

    @pl.when(cid == 0)
    def _init_x():
        for k in range(RPT // ZR):
            base = sid * RPT + k * ZR

            @pl.when(base + ZR <= N_NODES)
            def _cp():
                pltpu.sync_copy(x_hbm.at[pl.ds(base, ZR)], rows)
                pltpu.sync_copy(rows, acc.at[pl.ds(base, ZR)])

    plsc.subcore_barrier()

    @pl.loop(0, NCHUNK)
    def _chunk(i):
        base = wid * EPW + i * CH
        pltpu.sync_copy(src_hbm.at[pl.ds(base, CH)], idx_s)
        pltpu.sync_copy(dst_hbm.at[pl.ds(base, CH)], idx_d)
        pltpu.sync_copy(w_hbm.at[pl.ds(base, CH)], wv)
        pltpu.async_copy(x_hbm.at[idx_s], rows, gsem).wait()

        @pl.loop(0, CH // L)
        def _scale(g):
            vec = wv[pl.ds(g * L, L)]
            dvec = idx_d[pl.ds(g * L, L)]
            for r in range(L):
                e = g * L + r
                wb = jnp.full((L,), vec[r])
                for j in range(D // L):
                    sl = pl.ds(j * L, L)
                    rows[e, sl] = rows[e, sl] * wb
                d = dvec[r]
                deg[pl.ds(d, L)] = deg[pl.ds(d, L)] + jnp.where(lane == 0, wb, 0.0)

        pltpu.sync_copy(rows, acc.at[idx_d], add=True)

    plsc.subcore_barrier()

    for k in range(RPT // ZR):
        base = sid * RPT + k * ZR
        pltpu.sync_copy(acc.at[pl.ds(base, ZR)], rows)
        pltpu.sync_copy(rows, pooled_hbm.at[cid, pl.ds(base, ZR)])
    pltpu.sync_copy(deg, deg_hbm.at[pl.ds(wid * NPAD, NPAD)])


BLK = 1024


def _tc_body(pooled_ref, deg_ref, o_ref):
    p = pooled_ref[0] + pooled_ref[1]
    dtot = jnp.sum(deg_ref[...], axis=0)
    o_ref[...] = p * (1.0 / (1.0 + dtot))[:, None]


def kernel(x, edge_weight, edge_index):
    x = x.astype(jnp.float32)
    w = jnp.squeeze(edge_weight, -1).astype(jnp.float32)
    src = edge_index[0].astype(jnp.int32)
    dst = edge_index[1].astype(jnp.int32)

    sc = pl.kernel(
        _sc_body,
        out_type=[jax.ShapeDtypeStruct((NC, NPAD, D), jnp.float32),
                  jax.ShapeDtypeStruct((NW * NPAD,), jnp.float32)],
        mesh=plsc.VectorSubcoreMesh(core_axis_name="c", subcore_axis_name="s",
                                    num_cores=NC, num_subcores=NS),
        scratch_types=[
            pltpu.VMEM((CH,), jnp.int32),
            pltpu.VMEM((CH,), jnp.int32),
            pltpu.VMEM((CH,), jnp.float32),
            pltpu.VMEM((CH, D), jnp.float32),
            pltpu.VMEM((NPAD,), jnp.float32),
            pltpu.VMEM_SHARED((NPAD, D), jnp.float32),
            pltpu.SemaphoreType.DMA,
        ],
    )
    pooled, deg = sc(x, src, dst, w)
    deg = deg.reshape(NW, NPAD)

    out = pl.pallas_call(
        _tc_body,
        grid=(NPAD // BLK,),
        in_specs=[pl.BlockSpec((NC, BLK, D), lambda i: (0, i, 0)),
                  pl.BlockSpec((NW, BLK), lambda i: (0, i))],
        out_specs=pl.BlockSpec((BLK, D), lambda i: (i, 0)),
        out_shape=jax.ShapeDtypeStruct((NPAD, D), jnp.float32),
    )(pooled, deg)
    return out[:N_NODES]
